# baseline (speedup 1.0000x reference)
import jax
import jax.numpy as jnp
from jax import lax
from jax.experimental import pallas as pl
from jax.experimental.pallas import tpu as pltpu
from jax.experimental.pallas import tpu_sc as plsc

N = 10000
NPAD = 10112
RPT = NPAD // 16
E = 320000
CH = 128
WCH = 8
NWIN = 10
NCHD = NWIN * WCH
EPAD = 32 * NCHD * CH
NC, NS = 2, 16
BR = 1000


def _mesh():
    return plsc.VectorSubcoreMesh(
        core_axis_name="c", subcore_axis_name="s",
        num_cores=NC, num_subcores=NS)


def _zero_acc(zeros_hbm, acc, s):
    r0 = s * RPT
    pltpu.sync_copy(zeros_hbm, acc.at[pl.ds(r0, RPT)])


def _deg_body(dst_hbm, ones_hbm, zeros_hbm, deg_out, dst_v, ones_v, sem, acc):
    c = lax.axis_index("c")
    s = lax.axis_index("s")
    wid = s * NC + c
    _zero_acc(zeros_hbm, acc, s)
    pltpu.sync_copy(ones_hbm, ones_v)
    pltpu.sync_copy(dst_hbm.at[wid], dst_v)
    plsc.subcore_barrier()

    def body(j, _):
        pltpu.async_copy(ones_v, acc.at[dst_v.at[j]], sem, add=True)
        return 0
    lax.fori_loop(0, NCHD, body, 0)

    def drain(j, _):
        pltpu.make_async_copy(ones_v, acc.at[dst_v.at[j]], sem).wait()
        return 0
    lax.fori_loop(0, NCHD, drain, 0)

    plsc.subcore_barrier()
    r0 = s * RPT
    pltpu.sync_copy(acc.at[pl.ds(r0, RPT)], deg_out.at[c, pl.ds(r0, RPT)])


def _deg_call(dst32, ones_hbm, zeros_hbm):
    return pl.kernel(
        _deg_body,
        out_type=jax.ShapeDtypeStruct((NC, NPAD, 128), jnp.float32),
        mesh=_mesh(),
        scratch_types=[
            pltpu.VMEM((NCHD, CH), jnp.int32),
            pltpu.VMEM((CH, 128), jnp.float32),
            pltpu.SemaphoreType.DMA,
            pltpu.VMEM_SHARED((NPAD, 128), jnp.float32),
        ],
    )(dst32, ones_hbm, zeros_hbm)


def _spmm_body(zp_hbm, idx_hbm, zeros_hbm, aout,
               idx_v, buf, isem, sem, ssem, acc):
    c = lax.axis_index("c")
    s = lax.axis_index("s")
    wid = s * NC + c
    _zero_acc(zeros_hbm, acc, s)
    pltpu.async_copy(idx_hbm.at[wid, 0], idx_v.at[0], isem)
    plsc.subcore_barrier()

    def wbody(w, _):
        bw = lax.rem(w, 2)
        pltpu.make_async_copy(idx_hbm.at[wid, w], idx_v.at[bw], isem).wait()

        @pl.when(w < NWIN - 1)
        def _prefetch_idx():
            pltpu.async_copy(idx_hbm.at[wid, w + 1], idx_v.at[1 - bw], isem)

        pltpu.async_copy(zp_hbm.at[idx_v.at[bw, 0, 0]], buf.at[0], sem)

        def kbody(k, _):
            b = lax.rem(k, 2)
            pltpu.make_async_copy(
                zp_hbm.at[idx_v.at[bw, 0, k]], buf.at[b], sem).wait()

            @pl.when(k > 0)
            def _wait_prev_scatter():
                pltpu.make_async_copy(
                    buf.at[1 - b], acc.at[idx_v.at[bw, 1, k - 1]], ssem
                ).wait()

            @pl.when(k < WCH - 1)
            def _start_next_gather():
                pltpu.async_copy(
                    zp_hbm.at[idx_v.at[bw, 0, k + 1]], buf.at[1 - b], sem)

            pltpu.async_copy(
                buf.at[b], acc.at[idx_v.at[bw, 1, k]], ssem, add=True)
            return 0
        lax.fori_loop(0, WCH, kbody, 0)

        pltpu.make_async_copy(
            buf.at[lax.rem(WCH - 1, 2)],
            acc.at[idx_v.at[bw, 1, WCH - 1]], ssem).wait()
        return 0
    lax.fori_loop(0, NWIN, wbody, 0)

    plsc.subcore_barrier()
    r0 = s * RPT
    pltpu.sync_copy(acc.at[pl.ds(r0, RPT)], aout.at[c, pl.ds(r0, RPT)])


def _spmm_call(zp, idx5, zeros_hbm):
    return pl.kernel(
        _spmm_body,
        out_type=jax.ShapeDtypeStruct((NC, NPAD, 128), jnp.float32),
        mesh=_mesh(),
        scratch_types=[
            pltpu.VMEM((2, 2, WCH, CH), jnp.int32),
            pltpu.VMEM((2, CH, 128), jnp.float32),
            pltpu.SemaphoreType.DMA,
            pltpu.SemaphoreType.DMA,
            pltpu.SemaphoreType.DMA,
            pltpu.VMEM_SHARED((NPAD, 128), jnp.float32),
        ],
    )(zp, idx5, zeros_hbm)


def _dinv_of(deg_ref):
    deg = deg_ref[0, :, 0:1] + deg_ref[1, :, 0:1] + 1.0
    return lax.rsqrt(deg)


def _tc1_body(x_ref, w1_ref, deg_ref, zp_ref):
    dinv = _dinv_of(deg_ref)
    z = jnp.dot(x_ref[...], w1_ref[...], preferred_element_type=jnp.float32)
    zp_ref[...] = z * dinv


def _tc2_body(a1_ref, zp1_ref, deg_ref, b1_ref, w2_ref, zp_ref):
    dinv = _dinv_of(deg_ref)
    a = a1_ref[0] + a1_ref[1] + zp1_ref[...]
    h = jnp.maximum(a * dinv + b1_ref[...], 0.0)
    z = jnp.dot(h, w2_ref[...], preferred_element_type=jnp.float32)
    zd = z * dinv
    zp_ref[...] = jnp.concatenate(
        [zd, jnp.zeros((BR, 64), jnp.float32)], axis=1)


def _tc3_body(a2_ref, zp2_ref, deg_ref, b2_ref, wfc_ref, bfc_ref, out_ref):
    dinv = _dinv_of(deg_ref)
    a = (a2_ref[0] + a2_ref[1] + zp2_ref[...])[:, :64]
    h = jnp.maximum(a * dinv + b2_ref[...], 0.0)
    out_ref[...] = (
        jnp.dot(h, wfc_ref[...], preferred_element_type=jnp.float32)
        + bfc_ref[...])


_DEG_SPEC = pl.BlockSpec((NC, BR, 128), lambda i: (0, i, 0))
_ROW_SPEC = pl.BlockSpec((BR, 128), lambda i: (i, 0))
_PART_SPEC = pl.BlockSpec((NC, BR, 128), lambda i: (0, i, 0))


def _tc1(x, W1, degp):
    return pl.pallas_call(
        _tc1_body,
        grid=(N // BR,),
        in_specs=[
            _ROW_SPEC,
            pl.BlockSpec((128, 128), lambda i: (0, 0)),
            _DEG_SPEC,
        ],
        out_specs=_ROW_SPEC,
        out_shape=jax.ShapeDtypeStruct((NPAD, 128), jnp.float32),
    )(x, W1, degp)


def _tc2(a1, zp1, degp, b1r, W2):
    return pl.pallas_call(
        _tc2_body,
        grid=(N // BR,),
        in_specs=[
            _PART_SPEC,
            _ROW_SPEC,
            _DEG_SPEC,
            pl.BlockSpec((1, 128), lambda i: (0, 0)),
            pl.BlockSpec((128, 64), lambda i: (0, 0)),
        ],
        out_specs=_ROW_SPEC,
        out_shape=jax.ShapeDtypeStruct((NPAD, 128), jnp.float32),
    )(a1, zp1, degp, b1r, W2)


def _tc3(a2, zp2, degp, b2r, Wfc, bfcr):
    return pl.pallas_call(
        _tc3_body,
        grid=(N // BR,),
        in_specs=[
            _PART_SPEC,
            _ROW_SPEC,
            _DEG_SPEC,
            pl.BlockSpec((1, 64), lambda i: (0, 0)),
            pl.BlockSpec((64, 2), lambda i: (0, 0)),
            pl.BlockSpec((1, 2), lambda i: (0, 0)),
        ],
        out_specs=pl.BlockSpec((BR, 2), lambda i: (i, 0)),
        out_shape=jax.ShapeDtypeStruct((N, 2), jnp.float32),
    )(a2, zp2, degp, b2r, Wfc, bfcr)


def kernel(x, edge_index, W1, b1, W2, b2, Wfc, bfc):
    src = edge_index[0].astype(jnp.int32)
    dst = edge_index[1].astype(jnp.int32)
    pad = N + (jnp.arange(EPAD - E, dtype=jnp.int32) % (NPAD - N))
    srcp = jnp.concatenate([src, pad])
    dstp = jnp.concatenate([dst, pad])
    dst32 = dstp.reshape(32, NCHD, CH)
    idx5 = jnp.stack(
        [srcp.reshape(32, NWIN, WCH, CH), dstp.reshape(32, NWIN, WCH, CH)],
        axis=2)
    ones_hbm = jnp.ones((CH, 128), jnp.float32)
    zeros_hbm = jnp.zeros((RPT, 128), jnp.float32)

    degp = _deg_call(dst32, ones_hbm, zeros_hbm)
    zp1 = _tc1(x, W1, degp)
    a1 = _spmm_call(zp1, idx5, zeros_hbm)
    zp2 = _tc2(a1, zp1, degp, b1.reshape(1, -1), W2)
    a2 = _spmm_call(zp2, idx5, zeros_hbm)
    return _tc3(a2, zp2, degp, b2.reshape(1, -1), Wfc, bfc.reshape(1, -1))


# speedup vs baseline: 1.1374x; 1.1374x over previous
import jax
import jax.numpy as jnp
from jax import lax
from jax.experimental import pallas as pl
from jax.experimental.pallas import tpu as pltpu
from jax.experimental.pallas import tpu_sc as plsc

N = 10000
NPAD = 10112
RPT = NPAD // 16
E = 320000
CH = 128
WCH = 20
NWIN = 4
NCHD = NWIN * WCH
EPAD = 32 * NCHD * CH
NC, NS = 2, 16
BR = 1000


def _mesh():
    return plsc.VectorSubcoreMesh(
        core_axis_name="c", subcore_axis_name="s",
        num_cores=NC, num_subcores=NS)


def _zero_acc(zeros_hbm, acc, s):
    r0 = s * RPT
    pltpu.sync_copy(zeros_hbm, acc.at[pl.ds(r0, RPT)])


def _deg_body(dst_hbm, ones_hbm, zeros_hbm, deg_out, dst_v, ones_v, sem, acc):
    c = lax.axis_index("c")
    s = lax.axis_index("s")
    wid = s * NC + c
    r0 = s * RPT
    pltpu.sync_copy(zeros_hbm, acc.at[pl.ds(r0, RPT)])
    pltpu.sync_copy(ones_hbm, ones_v)
    pltpu.sync_copy(dst_hbm.at[wid], dst_v)
    plsc.subcore_barrier()

    def body(j, _):
        pltpu.async_copy(ones_v, acc.at[dst_v.at[j]], sem, add=True)
        return 0
    lax.fori_loop(0, NCHD, body, 0)

    def drain(j, _):
        pltpu.make_async_copy(ones_v, acc.at[dst_v.at[j]], sem).wait()
        return 0
    lax.fori_loop(0, NCHD, drain, 0)

    plsc.subcore_barrier()
    pltpu.sync_copy(acc.at[pl.ds(r0, RPT)], deg_out.at[c, pl.ds(r0, RPT)])


def _deg_call(dst32, ones_hbm, zeros_hbm):
    return pl.kernel(
        _deg_body,
        out_type=jax.ShapeDtypeStruct((NC, NPAD, 128), jnp.float32),
        mesh=_mesh(),
        scratch_types=[
            pltpu.VMEM((NCHD, CH), jnp.int32),
            pltpu.VMEM((CH, 128), jnp.float32),
            pltpu.SemaphoreType.DMA,
            pltpu.VMEM_SHARED((NPAD, 128), jnp.float32),
        ],
    )(dst32, ones_hbm, zeros_hbm)


def _spmm_body(zp_hbm, idx_hbm, zeros_hbm, aout, idx_v, buf, isem, sem, acc):
    c = lax.axis_index("c")
    s = lax.axis_index("s")
    wid = s * NC + c
    _zero_acc(zeros_hbm, acc, s)
    pltpu.async_copy(idx_hbm.at[wid, 0], idx_v.at[0], isem)
    plsc.subcore_barrier()

    def wbody(w, _):
        bw = lax.rem(w, 2)
        pltpu.make_async_copy(idx_hbm.at[wid, w], idx_v.at[bw], isem).wait()

        @pl.when(w < NWIN - 1)
        def _prefetch_idx():
            pltpu.async_copy(idx_hbm.at[wid, w + 1], idx_v.at[1 - bw], isem)

        def kbody(k, _):
            b = lax.rem(k, 2)

            @pl.when(k < WCH)
            def _start_gather():
                pltpu.async_copy(
                    zp_hbm.at[idx_v.at[bw, 0, k]], buf.at[b], sem)

            @pl.when(k > 0)
            def _scatter_prev():
                kp = k - 1
                pltpu.make_async_copy(
                    zp_hbm.at[idx_v.at[bw, 0, kp]], buf.at[1 - b], sem
                ).wait()
                pltpu.sync_copy(
                    buf.at[1 - b], acc.at[idx_v.at[bw, 1, kp]], add=True)
            return 0
        lax.fori_loop(0, WCH + 1, kbody, 0)
        return 0
    lax.fori_loop(0, NWIN, wbody, 0)

    plsc.subcore_barrier()
    r0 = s * RPT
    pltpu.sync_copy(acc.at[pl.ds(r0, RPT)], aout.at[c, pl.ds(r0, RPT)])


def _spmm_call(zp, idx5, zeros_hbm):
    return pl.kernel(
        _spmm_body,
        out_type=jax.ShapeDtypeStruct((NC, NPAD, 128), jnp.float32),
        mesh=_mesh(),
        scratch_types=[
            pltpu.VMEM((2, 2, WCH, CH), jnp.int32),
            pltpu.VMEM((2, CH, 128), jnp.float32),
            pltpu.SemaphoreType.DMA,
            pltpu.SemaphoreType.DMA,
            pltpu.VMEM_SHARED((NPAD, 128), jnp.float32),
        ],
    )(zp, idx5, zeros_hbm)


def _dinv_of(deg_ref):
    deg = deg_ref[0, :, 0:1] + deg_ref[1, :, 0:1] + 1.0
    return lax.rsqrt(deg)


def _tc1_body(x_ref, w1_ref, deg_ref, zp_ref, dn_ref):
    dinv = _dinv_of(deg_ref)
    z = jnp.dot(x_ref[...], w1_ref[...], preferred_element_type=jnp.float32)
    zp_ref[...] = z * dinv
    dn_ref[...] = jnp.broadcast_to(dinv, (BR, 8))


def _tc2_body(a1_ref, zp1_ref, dn_ref, b1_ref, w2_ref, zp_ref):
    dinv = dn_ref[:, 0:1]
    a = a1_ref[0] + a1_ref[1] + zp1_ref[...]
    h = jnp.maximum(a * dinv + b1_ref[...], 0.0)
    z = jnp.dot(h, w2_ref[...], preferred_element_type=jnp.float32)
    zd = z * dinv
    zp_ref[...] = jnp.concatenate(
        [zd, jnp.zeros((BR, 64), jnp.float32)], axis=1)


def _tc3_body(a2_ref, zp2_ref, dn_ref, b2_ref, wfc_ref, bfc_ref, out_ref):
    dinv = dn_ref[:, 0:1]
    a = (a2_ref[0] + a2_ref[1] + zp2_ref[...])[:, :64]
    h = jnp.maximum(a * dinv + b2_ref[...], 0.0)
    out_ref[...] = (
        jnp.dot(h, wfc_ref[...], preferred_element_type=jnp.float32)
        + bfc_ref[...])


_DEG_SPEC = pl.BlockSpec((NC, BR, 128), lambda i: (0, i, 0))
_ROW_SPEC = pl.BlockSpec((BR, 128), lambda i: (i, 0))
_PART_SPEC = pl.BlockSpec((NC, BR, 128), lambda i: (0, i, 0))


def _tc1(x, W1, degp):
    return pl.pallas_call(
        _tc1_body,
        grid=(N // BR,),
        in_specs=[
            _ROW_SPEC,
            pl.BlockSpec((128, 128), lambda i: (0, 0)),
            _DEG_SPEC,
        ],
        out_specs=(_ROW_SPEC, pl.BlockSpec((BR, 8), lambda i: (i, 0))),
        out_shape=(jax.ShapeDtypeStruct((NPAD, 128), jnp.float32),
                   jax.ShapeDtypeStruct((NPAD, 8), jnp.float32)),
    )(x, W1, degp)


def _tc2(a1, zp1, dn, b1r, W2):
    return pl.pallas_call(
        _tc2_body,
        grid=(N // BR,),
        in_specs=[
            _PART_SPEC,
            _ROW_SPEC,
            pl.BlockSpec((BR, 8), lambda i: (i, 0)),
            pl.BlockSpec((1, 128), lambda i: (0, 0)),
            pl.BlockSpec((128, 64), lambda i: (0, 0)),
        ],
        out_specs=_ROW_SPEC,
        out_shape=jax.ShapeDtypeStruct((NPAD, 128), jnp.float32),
    )(a1, zp1, dn, b1r, W2)


def _tc3(a2, zp2, dn, b2r, Wfc, bfcr):
    return pl.pallas_call(
        _tc3_body,
        grid=(N // BR,),
        in_specs=[
            _PART_SPEC,
            _ROW_SPEC,
            pl.BlockSpec((BR, 8), lambda i: (i, 0)),
            pl.BlockSpec((1, 64), lambda i: (0, 0)),
            pl.BlockSpec((64, 2), lambda i: (0, 0)),
            pl.BlockSpec((1, 2), lambda i: (0, 0)),
        ],
        out_specs=pl.BlockSpec((BR, 2), lambda i: (i, 0)),
        out_shape=jax.ShapeDtypeStruct((N, 2), jnp.float32),
    )(a2, zp2, dn, b2r, Wfc, bfcr)


def kernel(x, edge_index, W1, b1, W2, b2, Wfc, bfc):
    src = edge_index[0].astype(jnp.int32)
    dst = edge_index[1].astype(jnp.int32)
    pad = N + (jnp.arange(EPAD - E, dtype=jnp.int32) % (NPAD - N))
    srcp = jnp.concatenate([src, pad])
    dstp = jnp.concatenate([dst, pad])
    dst32 = dstp.reshape(32, NCHD, CH)
    idx5 = jnp.stack(
        [srcp.reshape(32, NWIN, WCH, CH), dstp.reshape(32, NWIN, WCH, CH)],
        axis=2)
    ones_hbm = jnp.ones((CH, 128), jnp.float32)
    zeros_hbm = jnp.zeros((RPT, 128), jnp.float32)

    degp = _deg_call(dst32, ones_hbm, zeros_hbm)
    zp1, dn = _tc1(x, W1, degp)
    a1 = _spmm_call(zp1, idx5, zeros_hbm)
    zp2 = _tc2(a1, zp1, dn, b1.reshape(1, -1), W2)
    a2 = _spmm_call(zp2, idx5, zeros_hbm)
    return _tc3(a2, zp2, dn, b2.reshape(1, -1), Wfc, bfc.reshape(1, -1))


# speedup vs baseline: 1.1587x; 1.0187x over previous
import jax
import jax.numpy as jnp
from jax import lax
from jax.experimental import pallas as pl
from jax.experimental.pallas import tpu as pltpu
from jax.experimental.pallas import tpu_sc as plsc

N = 10000
NPAD = 10112
RPT = NPAD // 16
E = 320000
CH = 128
WCH = 20
NWIN = 4
NCHD = NWIN * WCH
EPAD = 32 * NCHD * CH
NC, NS = 2, 16
BR = 1000


def _mesh():
    return plsc.VectorSubcoreMesh(
        core_axis_name="c", subcore_axis_name="s",
        num_cores=NC, num_subcores=NS)


def _deg_body(dst_hbm, ones_hbm, deg_out, dst_v, ones_v, zstage, sem, acc):
    c = lax.axis_index("c")
    s = lax.axis_index("s")
    wid = s * NC + c
    r0 = s * RPT
    zero16 = jnp.zeros((16,), jnp.float32)

    def zb(g, _):
        for k in range(8):
            zstage[g, pl.ds(k * 16, 16)] = zero16
        return 0
    lax.fori_loop(0, CH, zb, 0)
    for part in range(RPT // CH):
        pltpu.sync_copy(zstage, acc.at[pl.ds(r0 + part * CH, CH)])
    pltpu.sync_copy(
        zstage.at[pl.ds(0, RPT - (RPT // CH) * CH)],
        acc.at[pl.ds(r0 + (RPT // CH) * CH, RPT - (RPT // CH) * CH)])
    pltpu.sync_copy(ones_hbm, ones_v)
    pltpu.sync_copy(dst_hbm.at[wid], dst_v)
    plsc.subcore_barrier()

    def body(j, _):
        pltpu.async_copy(ones_v, acc.at[dst_v.at[j]], sem, add=True)
        return 0
    lax.fori_loop(0, NCHD, body, 0)

    def drain(j, _):
        pltpu.make_async_copy(ones_v, acc.at[dst_v.at[j]], sem).wait()
        return 0
    lax.fori_loop(0, NCHD, drain, 0)

    plsc.subcore_barrier()
    pltpu.sync_copy(acc.at[pl.ds(r0, RPT)], deg_out.at[c, pl.ds(r0, RPT)])


def _deg_call(dst32, ones_hbm):
    return pl.kernel(
        _deg_body,
        out_type=jax.ShapeDtypeStruct((NC, NPAD, 128), jnp.float32),
        mesh=_mesh(),
        scratch_types=[
            pltpu.VMEM((NCHD, CH), jnp.int32),
            pltpu.VMEM((CH, 128), jnp.float32),
            pltpu.VMEM((CH, 128), jnp.float32),
            pltpu.SemaphoreType.DMA,
            pltpu.VMEM_SHARED((NPAD, 128), jnp.float32),
        ],
    )(dst32, ones_hbm)


def _spmm_body(zp_hbm, idx_hbm, aout, idx_v, buf, isem, sem, acc):
    c = lax.axis_index("c")
    s = lax.axis_index("s")
    wid = s * NC + c
    r0 = s * RPT
    pltpu.async_copy(idx_hbm.at[wid, 0], idx_v.at[0], isem)

    @pl.when(c == 0)
    def _init_selfloop():
        pltpu.sync_copy(zp_hbm.at[pl.ds(r0, RPT)], acc.at[pl.ds(r0, RPT)])

    @pl.when(c == 1)
    def _init_zero():
        zero16 = jnp.zeros((16,), jnp.float32)

        def zb(g, _):
            for k in range(8):
                buf[0, g, pl.ds(k * 16, 16)] = zero16
            return 0
        lax.fori_loop(0, CH, zb, 0)
        for part in range(RPT // CH):
            pltpu.sync_copy(
                buf.at[0], acc.at[pl.ds(r0 + part * CH, CH)])
        pltpu.sync_copy(
            buf.at[0, pl.ds(0, RPT - (RPT // CH) * CH)],
            acc.at[pl.ds(r0 + (RPT // CH) * CH, RPT - (RPT // CH) * CH)])
    plsc.subcore_barrier()

    def wbody(w, _):
        bw = lax.rem(w, 2)
        pltpu.make_async_copy(idx_hbm.at[wid, w], idx_v.at[bw], isem).wait()

        @pl.when(w < NWIN - 1)
        def _prefetch_idx():
            pltpu.async_copy(idx_hbm.at[wid, w + 1], idx_v.at[1 - bw], isem)

        def kbody(k, _):
            b = lax.rem(k, 2)

            @pl.when(k < WCH)
            def _start_gather():
                pltpu.async_copy(
                    zp_hbm.at[idx_v.at[bw, 0, k]], buf.at[b], sem)

            @pl.when(k > 0)
            def _scatter_prev():
                kp = k - 1
                pltpu.make_async_copy(
                    zp_hbm.at[idx_v.at[bw, 0, kp]], buf.at[1 - b], sem
                ).wait()
                pltpu.sync_copy(
                    buf.at[1 - b], acc.at[idx_v.at[bw, 1, kp]], add=True)
            return 0
        lax.fori_loop(0, WCH + 1, kbody, 0)
        return 0
    lax.fori_loop(0, NWIN, wbody, 0)

    plsc.subcore_barrier()
    r0 = s * RPT
    pltpu.sync_copy(acc.at[pl.ds(r0, RPT)], aout.at[c, pl.ds(r0, RPT)])


def _spmm_call(zp, idx5):
    return pl.kernel(
        _spmm_body,
        out_type=jax.ShapeDtypeStruct((NC, NPAD, 128), jnp.float32),
        mesh=_mesh(),
        scratch_types=[
            pltpu.VMEM((2, 2, WCH, CH), jnp.int32),
            pltpu.VMEM((2, CH, 128), jnp.float32),
            pltpu.SemaphoreType.DMA,
            pltpu.SemaphoreType.DMA,
            pltpu.VMEM_SHARED((NPAD, 128), jnp.float32),
        ],
    )(zp, idx5)


def _dinv_of(deg_ref):
    deg = deg_ref[0, :, 0:1] + deg_ref[1, :, 0:1] + 1.0
    return lax.rsqrt(deg)


def _tc1_body(x_ref, w1_ref, deg_ref, zp_ref, dn_ref):
    dinv = _dinv_of(deg_ref)
    z = jnp.dot(x_ref[...], w1_ref[...], preferred_element_type=jnp.float32)
    zp_ref[...] = z * dinv
    dn_ref[...] = jnp.broadcast_to(dinv, (BR, 8))


def _tc2_body(a1_ref, dn_ref, b1_ref, w2_ref, zp_ref):
    dinv = dn_ref[:, 0:1]
    a = a1_ref[0] + a1_ref[1]
    h = jnp.maximum(a * dinv + b1_ref[...], 0.0)
    z = jnp.dot(h, w2_ref[...], preferred_element_type=jnp.float32)
    zd = z * dinv
    zp_ref[...] = jnp.concatenate(
        [zd, jnp.zeros((BR, 64), jnp.float32)], axis=1)


def _tc3_body(a2_ref, dn_ref, b2_ref, wfc_ref, bfc_ref, out_ref):
    dinv = dn_ref[:, 0:1]
    a = (a2_ref[0] + a2_ref[1])[:, :64]
    h = jnp.maximum(a * dinv + b2_ref[...], 0.0)
    out_ref[...] = (
        jnp.dot(h, wfc_ref[...], preferred_element_type=jnp.float32)
        + bfc_ref[...])


_DEG_SPEC = pl.BlockSpec((NC, BR, 128), lambda i: (0, i, 0))
_ROW_SPEC = pl.BlockSpec((BR, 128), lambda i: (i, 0))
_PART_SPEC = pl.BlockSpec((NC, BR, 128), lambda i: (0, i, 0))


def _tc1(x, W1, degp):
    return pl.pallas_call(
        _tc1_body,
        grid=(N // BR,),
        in_specs=[
            _ROW_SPEC,
            pl.BlockSpec((128, 128), lambda i: (0, 0)),
            _DEG_SPEC,
        ],
        out_specs=(_ROW_SPEC, pl.BlockSpec((BR, 8), lambda i: (i, 0))),
        out_shape=(jax.ShapeDtypeStruct((NPAD, 128), jnp.float32),
                   jax.ShapeDtypeStruct((NPAD, 8), jnp.float32)),
    )(x, W1, degp)


def _tc2(a1, dn, b1r, W2):
    return pl.pallas_call(
        _tc2_body,
        grid=(N // BR,),
        in_specs=[
            _PART_SPEC,
            pl.BlockSpec((BR, 8), lambda i: (i, 0)),
            pl.BlockSpec((1, 128), lambda i: (0, 0)),
            pl.BlockSpec((128, 64), lambda i: (0, 0)),
        ],
        out_specs=_ROW_SPEC,
        out_shape=jax.ShapeDtypeStruct((NPAD, 128), jnp.float32),
    )(a1, dn, b1r, W2)


def _tc3(a2, dn, b2r, Wfc, bfcr):
    return pl.pallas_call(
        _tc3_body,
        grid=(N // BR,),
        in_specs=[
            _PART_SPEC,
            pl.BlockSpec((BR, 8), lambda i: (i, 0)),
            pl.BlockSpec((1, 64), lambda i: (0, 0)),
            pl.BlockSpec((64, 2), lambda i: (0, 0)),
            pl.BlockSpec((1, 2), lambda i: (0, 0)),
        ],
        out_specs=pl.BlockSpec((BR, 2), lambda i: (i, 0)),
        out_shape=jax.ShapeDtypeStruct((N, 2), jnp.float32),
    )(a2, dn, b2r, Wfc, bfcr)


def kernel(x, edge_index, W1, b1, W2, b2, Wfc, bfc):
    src = edge_index[0].astype(jnp.int32)
    dst = edge_index[1].astype(jnp.int32)
    pad = N + (jnp.arange(EPAD - E, dtype=jnp.int32) % (NPAD - N))
    srcp = jnp.concatenate([src, pad])
    dstp = jnp.concatenate([dst, pad])
    dst32 = dstp.reshape(32, NCHD, CH)
    idx5 = jnp.stack(
        [srcp.reshape(32, NWIN, WCH, CH), dstp.reshape(32, NWIN, WCH, CH)],
        axis=2)
    ones_hbm = jnp.ones((CH, 128), jnp.float32)

    degp = _deg_call(dst32, ones_hbm)
    zp1, dn = _tc1(x, W1, degp)
    a1 = _spmm_call(zp1, idx5)
    zp2 = _tc2(a1, dn, b1.reshape(1, -1), W2)
    a2 = _spmm_call(zp2, idx5)
    return _tc3(a2, dn, b2.reshape(1, -1), Wfc, bfc.reshape(1, -1))
